# baseline (speedup 1.0000x reference)
import functools

import jax
import jax.numpy as jnp
from jax import lax
from jax.experimental import pallas as pl
from jax.experimental.pallas import tpu as pltpu, tpu_sc as plsc

B, T, D, NBINS = 4096, 50, 128, 256
BT = B * T
NCOLG = 4
NPOSG = 8
COLS = D // NCOLG
PPT = BT // NPOSG
NBLK_POS = 640
NBLK = PPT // NBLK_POS
NGRP = NBLK_POS // 16

_mesh = plsc.VectorSubcoreMesh(core_axis_name="c", subcore_axis_name="s")


@functools.partial(
    pl.kernel,
    mesh=_mesh,
    out_type=jax.ShapeDtypeStruct((BT, D), jnp.float32),
    compiler_params=pltpu.CompilerParams(
        use_tc_tiling_on_sc=False, needs_layout_passes=False
    ),
    scratch_types=[
        pltpu.VMEM((6 * NBINS * COLS,), jnp.float32),
        pltpu.VMEM((2 * COLS,), jnp.float32),
        pltpu.VMEM((2, 7, NBLK_POS), jnp.int32),
        pltpu.VMEM((2, NBLK_POS, COLS), jnp.float32),
        pltpu.SemaphoreType.DMA,
        pltpu.SemaphoreType.DMA,
        pltpu.SemaphoreType.DMA,
    ],
)
def _encode(idx_t, warm, wgb, out_hbm, tab_v, wg_v, act_v, out_v,
            sem_in, sem_o0, sem_o1):
    wid = lax.axis_index("s") * 2 + lax.axis_index("c")
    colg = wid % NCOLG
    posg = wid // NCOLG
    c0 = colg * COLS
    p0 = posg * PPT

    pltpu.sync_copy(warm.at[colg], tab_v)
    pltpu.sync_copy(wgb.at[colg], wg_v)

    w0 = [wg_v[pl.ds(0, 16)], wg_v[pl.ds(16, 16)]]
    w1 = [wg_v[pl.ds(32, 16)], wg_v[pl.ds(48, 16)]]
    iota = [lax.iota(jnp.int32, 16), lax.iota(jnp.int32, 16) + 16]

    sem_o = [sem_o0, sem_o1]

    def cp_in(b, k):
        pb = p0 + b * NBLK_POS
        return pltpu.make_async_copy(
            idx_t.at[:, pl.ds(pb, NBLK_POS)], act_v.at[k], sem_in)

    def cp_out(b, k):
        pb = p0 + b * NBLK_POS
        return pltpu.make_async_copy(
            out_v.at[k],
            out_hbm.at[pl.ds(pb, NBLK_POS), pl.ds(c0, COLS)], sem_o[k])

    cp_in(0, 0).start()

    def block_pair(j, carry):
        for k in range(2):
            b = j * 2 + k
            cp_in(b, k).wait()

            @pl.when(b + 1 < NBLK)
            def _():
                cp_in(b + 1, 1 - k).start()

            @pl.when(b >= 2)
            def _():
                cp_out(b - 2, k).wait()

            @plsc.parallel_loop(0, NBLK_POS, step=16, unroll=8)
            def group_body(s):
                av = [act_v[k, i, pl.ds(s, 16)] for i in range(6)]
                gv = act_v[k, 6, pl.ds(s, 16)]
                for p in range(16):
                    pv = jnp.full((16,), p, jnp.int32)
                    m = gv.at[pv].get(mode="promise_in_bounds") > 0
                    ra = [av[i].at[pv].get(mode="promise_in_bounds")
                          for i in range(6)]
                    for h in range(2):
                        t = [plsc.load_gather(tab_v, [ra[i] + iota[h]])
                             for i in range(6)]
                        acc = (jnp.where(m, w1[h], w0[h]) + t[0]) + (t[1] + t[2])
                        acc = acc + ((t[3] + t[4]) + t[5])
                        out_v[k, s + p, pl.ds(h * 16, 16)] = acc

            cp_out(b, k).start()
        return carry

    lax.fori_loop(0, NBLK // 2, block_pair, 0)
    cp_out(NBLK - 2, 0).wait()
    cp_out(NBLK - 1, 1).wait()


def kernel(actions, base, W0, W1, W2, W3, W4, W5, Wg):
    acts = actions.reshape(BT, 7).T
    arm_idx = acts[:6] * COLS + (jnp.arange(6, dtype=jnp.int32) * (NBINS * COLS))[:, None]
    idx_t = jnp.concatenate([arm_idx, acts[6:]], axis=0)
    warm = jnp.stack([W0, W1, W2, W3, W4, W5])
    warm = warm.reshape(6, NBINS, NCOLG, COLS)
    warm = warm.transpose(2, 0, 1, 3).reshape(NCOLG, 6 * NBINS * COLS)
    wgb = Wg + base[None, :]
    wgb = wgb.reshape(2, NCOLG, COLS).transpose(1, 0, 2).reshape(NCOLG, 2 * COLS)
    out = _encode(idx_t, warm, wgb)
    return out.reshape(B, T, 1, D)


# speedup vs baseline: 2.8504x; 2.8504x over previous
import functools

import jax
import jax.numpy as jnp
from jax import lax
from jax.experimental import pallas as pl
from jax.experimental.pallas import tpu as pltpu, tpu_sc as plsc

B, T, D, NBINS = 4096, 50, 128, 256
BT = B * T
NCOLG = 4
NPOSG = 8
COLS = D // NCOLG
PPT = BT // NPOSG
NBLK_POS = 640
NBLK = PPT // NBLK_POS
NGRP = NBLK_POS // 16

_mesh = plsc.VectorSubcoreMesh(core_axis_name="c", subcore_axis_name="s")


@functools.partial(
    pl.kernel,
    mesh=_mesh,
    out_type=jax.ShapeDtypeStruct((BT, D), jnp.float32),
    compiler_params=pltpu.CompilerParams(
        use_tc_tiling_on_sc=False, needs_layout_passes=False
    ),
    scratch_types=[
        pltpu.VMEM((6 * NBINS * COLS,), jnp.float32),
        pltpu.VMEM((2 * COLS,), jnp.float32),
        pltpu.VMEM((2, 7, NBLK_POS), jnp.int32),
        pltpu.VMEM((2, NBLK_POS, COLS), jnp.float32),
        pltpu.SemaphoreType.DMA,
        pltpu.SemaphoreType.DMA,
        pltpu.SemaphoreType.DMA,
    ],
)
def _encode(idx_t, warm, wgb, out_hbm, tab_v, wg_v, act_v, out_v,
            sem_in, sem_o0, sem_o1):
    wid = lax.axis_index("s") * 2 + lax.axis_index("c")
    colg = wid % NCOLG
    posg = wid // NCOLG
    c0 = colg * COLS
    p0 = posg * PPT

    pltpu.sync_copy(warm.at[colg], tab_v)
    pltpu.sync_copy(wgb.at[colg], wg_v)

    w0 = [wg_v[pl.ds(0, 16)], wg_v[pl.ds(16, 16)]]
    w1 = [wg_v[pl.ds(32, 16)], wg_v[pl.ds(48, 16)]]
    iota = [lax.iota(jnp.int32, 16), lax.iota(jnp.int32, 16) + 16]

    sem_o = [sem_o0, sem_o1]

    def cp_in(b, k):
        pb = p0 + b * NBLK_POS
        return pltpu.make_async_copy(
            idx_t.at[:, pl.ds(pb, NBLK_POS)], act_v.at[k], sem_in)

    def cp_out(b, k):
        pb = p0 + b * NBLK_POS
        return pltpu.make_async_copy(
            out_v.at[k],
            out_hbm.at[pl.ds(pb, NBLK_POS), pl.ds(c0, COLS)], sem_o[k])

    cp_in(0, 0).start()

    def block_pair(j, carry):
        for k in range(2):
            b = j * 2 + k
            cp_in(b, k).wait()

            @pl.when(b + 1 < NBLK)
            def _():
                cp_in(b + 1, 1 - k).start()

            @pl.when(b >= 2)
            def _():
                cp_out(b - 2, k).wait()

            @plsc.parallel_loop(0, NBLK_POS, step=16, unroll=4)
            def group_body(s):
                av = [act_v[k, i, pl.ds(s, 16)] for i in range(6)]
                gv = act_v[k, 6, pl.ds(s, 16)]
                for p in range(16):
                    pv = jnp.full((16,), p, jnp.int32)
                    m = gv.at[pv].get(mode="promise_in_bounds") > 0
                    ra = [av[i].at[pv].get(mode="promise_in_bounds")
                          for i in range(6)]
                    for h in range(2):
                        t = [plsc.load_gather(tab_v, [ra[i] + iota[h]])
                             for i in range(6)]
                        acc = (jnp.where(m, w1[h], w0[h]) + t[0]) + (t[1] + t[2])
                        acc = acc + ((t[3] + t[4]) + t[5])
                        out_v[k, s + p, pl.ds(h * 16, 16)] = acc

            cp_out(b, k).start()
        return carry

    lax.fori_loop(0, NBLK // 2, block_pair, 0)
    cp_out(NBLK - 2, 0).wait()
    cp_out(NBLK - 1, 1).wait()


def kernel(actions, base, W0, W1, W2, W3, W4, W5, Wg):
    acts = actions.reshape(BT, 7).T
    arm_idx = acts[:6] * COLS + (jnp.arange(6, dtype=jnp.int32) * (NBINS * COLS))[:, None]
    idx_t = jnp.concatenate([arm_idx, acts[6:]], axis=0)
    warm = jnp.stack([W0, W1, W2, W3, W4, W5])
    warm = warm.reshape(6, NBINS, NCOLG, COLS)
    warm = warm.transpose(2, 0, 1, 3).reshape(NCOLG, 6 * NBINS * COLS)
    wgb = Wg + base[None, :]
    wgb = wgb.reshape(2, NCOLG, COLS).transpose(1, 0, 2).reshape(NCOLG, 2 * COLS)
    out = _encode(idx_t, warm, wgb)
    return out.reshape(B, T, 1, D)
